# Initial kernel scaffold; baseline (speedup 1.0000x reference)
import jax
import jax.numpy as jnp
from jax.experimental import pallas as pl


def kernel(x, weight):
    raise NotImplementedError("write your pallas kernel here")


# speedup vs baseline: 1.4729x; 1.4729x over previous
import functools

import jax
import jax.numpy as jnp
import numpy as np
from jax import lax
from jax.experimental import pallas as pl
from jax.experimental.pallas import tpu as pltpu
from jax.experimental.pallas import tpu_sc as plsc

NUM_EMB = 1000000
DIM = 64
QMAX = 127.0


_ROWS_PER_BLK = 8000


def _scale_body(w_ref, out_ref):
    i = pl.program_id(0)
    m = jnp.max(jnp.abs(w_ref[...]))

    @pl.when(i == 0)
    def _init():
        out_ref[0, 0] = m

    @pl.when(i > 0)
    def _acc():
        out_ref[0, 0] = jnp.maximum(out_ref[0, 0], m)

    @pl.when(i == pl.num_programs(0) - 1)
    def _fin():
        out_ref[0, 0] = jnp.maximum(out_ref[0, 0] / QMAX, 1e-8)


def _compute_scale(weight):
    return pl.pallas_call(
        _scale_body,
        grid=(NUM_EMB // _ROWS_PER_BLK,),
        in_specs=[pl.BlockSpec((_ROWS_PER_BLK, DIM), lambda i: (i, 0))],
        out_specs=pl.BlockSpec(memory_space=pltpu.SMEM),
        out_shape=jax.ShapeDtypeStruct((1, 1), jnp.float32),
    )(weight)


_B = 16384 * 50
_NW = 32
_B_PER_W = _B // _NW
_CHUNK = 800
_NCHUNK = _B_PER_W // _CHUNK

_SIGN_MASK = np.uint32(0x80000000)
_MAGIC_BITS = np.uint32(0x4B000000)


def _gather_fq(table, idx_flat, scale_vec):
    mesh = plsc.VectorSubcoreMesh(core_axis_name="c", subcore_axis_name="s")

    @functools.partial(
        pl.kernel,
        mesh=mesh,
        out_type=jax.ShapeDtypeStruct((_B, DIM), jnp.float32),
        scratch_types=[
            pltpu.VMEM((2, _CHUNK), jnp.int32),
            [pltpu.VMEM((_CHUNK, DIM), jnp.float32) for _ in range(2)],
            pltpu.VMEM((16,), jnp.float32),
            [pltpu.SemaphoreType.DMA for _ in range(2)],
            [pltpu.SemaphoreType.DMA for _ in range(2)],
        ],
        compiler_params=pltpu.CompilerParams(use_tc_tiling_on_sc=False,
                                             needs_layout_passes=False),
    )
    def k(table_hbm, idx_hbm, scale_hbm, out_hbm, idx_v, rows_v, scale_v,
          sem_g, sem_o):
        wid = lax.axis_index("s") * 2 + lax.axis_index("c")
        base = wid * _B_PER_W
        pltpu.sync_copy(scale_hbm, scale_v)
        s = scale_v[...]
        rs = 1.0 / s

        def dequant(buf):
            def row_body(r, _):
                for c in range(DIM // 16):
                    v = buf[r, pl.ds(c * 16, 16)]
                    t = v * rs
                    tb = plsc.bitcast(t, jnp.uint32)
                    csign = plsc.bitcast((tb & _SIGN_MASK) | _MAGIC_BITS,
                                         jnp.float32)
                    q = (t + csign) - csign
                    buf[r, pl.ds(c * 16, 16)] = q * s
                return 0

            lax.fori_loop(0, _CHUNK, row_body, 0, unroll=False)

        def start_gather(b, j):
            off = base + j * _CHUNK
            pltpu.sync_copy(idx_hbm.at[pl.ds(off, _CHUNK)], idx_v.at[b])
            pltpu.async_copy(table_hbm.at[idx_v.at[b]], rows_v[b], sem_g[b])

        for b in range(2):
            start_gather(b, b)

        def pair_body(p, _):
            for b in range(2):
                j = 2 * p + b
                off = base + j * _CHUNK
                pltpu.make_async_copy(table_hbm.at[idx_v.at[b]], rows_v[b],
                                      sem_g[b]).wait()
                dequant(rows_v[b])
                pltpu.async_copy(rows_v[b], out_hbm.at[pl.ds(off, _CHUNK)],
                                 sem_o[b])

                @pl.when(p < _NCHUNK // 2 - 1)
                def _prefetch():
                    pltpu.make_async_copy(rows_v[b],
                                          out_hbm.at[pl.ds(base, _CHUNK)],
                                          sem_o[b]).wait()
                    start_gather(b, j + 2)

            return 0

        lax.fori_loop(0, _NCHUNK // 2, pair_body, 0, unroll=False)

        for b in range(2):
            pltpu.make_async_copy(rows_v[b], out_hbm.at[pl.ds(base, _CHUNK)],
                                  sem_o[b]).wait()

    return k(table, idx_flat, scale_vec)


def kernel(x, weight):
    scale = _compute_scale(weight)
    scale_vec = jnp.broadcast_to(scale.reshape(()), (16,))
    out = _gather_fq(weight, x.reshape(-1), scale_vec)
    return out.reshape(x.shape[0], x.shape[1], DIM)
